# baseline (speedup 1.0000x reference)
import functools

import jax
import jax.numpy as jnp
from jax import lax
from jax.experimental import pallas as pl
from jax.experimental.pallas import tpu as pltpu
from jax.experimental.pallas import tpu_sc as plsc

NUM_TOKENS = 4096 * 200
DIM = 128
NUM_WORKERS = 32
ROWS_PER_WORKER = NUM_TOKENS // NUM_WORKERS
GATHER = 128
NGATHER = 2
CHUNK = GATHER * NGATHER
NCHUNKS = ROWS_PER_WORKER // CHUNK
NBUF = 2
NGROUPS = NCHUNKS // NBUF
IDXROWS = ROWS_PER_WORKER // GATHER

_mesh = plsc.VectorSubcoreMesh(core_axis_name="c", subcore_axis_name="s")


@functools.partial(
    pl.kernel,
    mesh=_mesh,
    out_type=jax.ShapeDtypeStruct((NUM_TOKENS, DIM), jnp.float32),
    scratch_types=[
        pltpu.VMEM((IDXROWS, GATHER), jnp.int32),
        pltpu.VMEM((NBUF, CHUNK, DIM), jnp.float32),
    ] + [pltpu.SemaphoreType.DMA] * (2 * NBUF),
)
def _embed(ids_hbm, table_hbm, out_hbm, idx_v, rows_v, *sems):
    gsem = sems[:NBUF]
    ssem = sems[NBUF:]
    wid = lax.axis_index("s") * 2 + lax.axis_index("c")
    idx_base = wid * IDXROWS
    tok_base = idx_base * GATHER

    pltpu.sync_copy(ids_hbm.at[pl.ds(idx_base, IDXROWS)], idx_v)

    def fire_gathers(ci, slot):
        for j in range(NGATHER):
            pltpu.async_copy(
                table_hbm.at[idx_v.at[ci * NGATHER + j]],
                rows_v.at[slot].at[pl.ds(j * GATHER, GATHER)],
                gsem[slot])

    def wait_gathers(ci, slot):
        for j in range(NGATHER):
            pltpu.make_async_copy(
                table_hbm.at[idx_v.at[ci * NGATHER + j]],
                rows_v.at[slot].at[pl.ds(j * GATHER, GATHER)],
                gsem[slot]).wait()

    def fire_store(ci, slot):
        pltpu.async_copy(
            rows_v.at[slot],
            out_hbm.at[pl.ds(tok_base + ci * CHUNK, CHUNK)],
            ssem[slot])

    def wait_store(ci, slot):
        pltpu.make_async_copy(
            rows_v.at[slot],
            out_hbm.at[pl.ds(tok_base + ci * CHUNK, CHUNK)],
            ssem[slot]).wait()

    for b in range(NBUF - 1):
        fire_gathers(b, b)

    def group(g, _):
        for b in range(NBUF):
            ci = g * NBUF + b
            nslot = (b + NBUF - 1) % NBUF
            nci = ci + NBUF - 1

            @pl.when(jnp.logical_and(nci < NCHUNKS, ci >= 1))
            def _():
                wait_store(ci - 1, nslot)

            @pl.when(nci < NCHUNKS)
            def _():
                fire_gathers(nci, nslot)

            wait_gathers(ci, b)
            fire_store(ci, b)
        return 0

    lax.fori_loop(0, NGROUPS, group, 0)

    for b in range(NBUF):
        ci = NCHUNKS - NBUF + b
        wait_store(ci, b)


def kernel(token_ids, weight):
    ids = token_ids.astype(jnp.int32).reshape(NUM_TOKENS // GATHER, GATHER)
    out = _embed(ids, weight)
    return out.reshape(token_ids.shape + (DIM,))


# speedup vs baseline: 1.0057x; 1.0057x over previous
import functools

import jax
import jax.numpy as jnp
from jax import lax
from jax.experimental import pallas as pl
from jax.experimental.pallas import tpu as pltpu
from jax.experimental.pallas import tpu_sc as plsc

NUM_TOKENS = 4096 * 200
DIM = 128
NUM_WORKERS = 32
ROWS_PER_WORKER = NUM_TOKENS // NUM_WORKERS
GATHER = 128
NGATHER = 1
CHUNK = GATHER * NGATHER
NCHUNKS = ROWS_PER_WORKER // CHUNK
NBUF = 4
NGROUPS = NCHUNKS // NBUF
IDXROWS = ROWS_PER_WORKER // GATHER

_mesh = plsc.VectorSubcoreMesh(core_axis_name="c", subcore_axis_name="s")


@functools.partial(
    pl.kernel,
    mesh=_mesh,
    out_type=jax.ShapeDtypeStruct((NUM_TOKENS, DIM), jnp.float32),
    scratch_types=[
        pltpu.VMEM((IDXROWS, GATHER), jnp.int32),
        pltpu.VMEM((NBUF, CHUNK, DIM), jnp.float32),
    ] + [pltpu.SemaphoreType.DMA] * (2 * NBUF),
)
def _embed(ids_hbm, table_hbm, out_hbm, idx_v, rows_v, *sems):
    gsem = sems[:NBUF]
    ssem = sems[NBUF:]
    wid = lax.axis_index("s") * 2 + lax.axis_index("c")
    idx_base = wid * IDXROWS
    tok_base = idx_base * GATHER

    pltpu.sync_copy(ids_hbm.at[pl.ds(idx_base, IDXROWS)], idx_v)

    def fire_gathers(ci, slot):
        for j in range(NGATHER):
            pltpu.async_copy(
                table_hbm.at[idx_v.at[ci * NGATHER + j]],
                rows_v.at[slot].at[pl.ds(j * GATHER, GATHER)],
                gsem[slot])

    def wait_gathers(ci, slot):
        for j in range(NGATHER):
            pltpu.make_async_copy(
                table_hbm.at[idx_v.at[ci * NGATHER + j]],
                rows_v.at[slot].at[pl.ds(j * GATHER, GATHER)],
                gsem[slot]).wait()

    def fire_store(ci, slot):
        pltpu.async_copy(
            rows_v.at[slot],
            out_hbm.at[pl.ds(tok_base + ci * CHUNK, CHUNK)],
            ssem[slot])

    def wait_store(ci, slot):
        pltpu.make_async_copy(
            rows_v.at[slot],
            out_hbm.at[pl.ds(tok_base + ci * CHUNK, CHUNK)],
            ssem[slot]).wait()

    for b in range(NBUF - 1):
        fire_gathers(b, b)

    def group(g, _):
        for b in range(NBUF):
            ci = g * NBUF + b
            nslot = (b + NBUF - 1) % NBUF
            nci = ci + NBUF - 1

            @pl.when(jnp.logical_and(nci < NCHUNKS, ci >= 1))
            def _():
                wait_store(ci - 1, nslot)

            @pl.when(nci < NCHUNKS)
            def _():
                fire_gathers(nci, nslot)

            wait_gathers(ci, b)
            fire_store(ci, b)
        return 0

    lax.fori_loop(0, NGROUPS, group, 0)

    for b in range(NBUF):
        ci = NCHUNKS - NBUF + b
        wait_store(ci, b)


def kernel(token_ids, weight):
    ids = token_ids.astype(jnp.int32).reshape(NUM_TOKENS // GATHER, GATHER)
    out = _embed(ids, weight)
    return out.reshape(token_ids.shape + (DIM,))
